# baseline (speedup 1.0000x reference)
import jax
import jax.numpy as jnp
from jax.experimental import pallas as pl


def _conv2d(x, W, b=None):
    y = jnp.einsum('oi,binx->bonx', W, x)
    if b is not None:
        y = y + b[None, :, None, None]
    return y


def _bn(y, g, be):
    mean = y.mean(axis=(0, 2, 3), keepdims=True)
    var = y.var(axis=(0, 2, 3), keepdims=True)
    y = (y - mean) / jnp.sqrt(var + 1e-5)
    return g[None, :, None, None] * y + be[None, :, None, None]


def _conv_bn(x, p, act=True):
    y = _bn(_conv2d(x, p['W'], p['b']), p['g'], p['be'])
    return jax.nn.relu(y) if act else y


def _gather_neighbour(pc, neigh_idx):
    b, n, d = pc.shape
    k = neigh_idx.shape[-1]
    idx = neigh_idx.reshape(b, n * k)
    feats = jax.vmap(lambda p, i: p[i])(pc, idx)
    return feats.reshape(b, n, k, d)


def _rel_pos_enc(xyz, neigh_idx):
    nxyz = _gather_neighbour(xyz, neigh_idx)
    tile = jnp.broadcast_to(xyz[:, :, None, :], nxyz.shape)
    rel = tile - nxyz
    dis = jnp.sqrt(jnp.sum(rel ** 2, axis=-1, keepdims=True) + 1e-12)
    return jnp.concatenate([dis, rel, tile, nxyz], axis=-1)


def _att_pool(fs, p):
    att = _conv2d(fs, p['Wfc'])
    scores = jax.nn.softmax(att, axis=3)
    agg = jnp.sum(fs * scores, axis=3, keepdims=True)
    return _conv_bn(agg, p['mlp'], act=True)


def _building_block(xyz, feature, neigh_idx, p):
    f_xyz = _rel_pos_enc(xyz, neigh_idx)
    f_xyz = jnp.transpose(f_xyz, (0, 3, 1, 2))
    f_xyz = _conv_bn(f_xyz, p['mlp1'])
    f_nb = _gather_neighbour(jnp.transpose(feature[..., 0], (0, 2, 1)), neigh_idx)
    f_nb = jnp.transpose(f_nb, (0, 3, 1, 2))
    f_cat = jnp.concatenate([f_nb, f_xyz], axis=1)
    f_agg = _att_pool(f_cat, p['att1'])
    f_xyz = _conv_bn(f_xyz, p['mlp2'])
    f_nb = _gather_neighbour(jnp.transpose(f_agg[..., 0], (0, 2, 1)), neigh_idx)
    f_nb = jnp.transpose(f_nb, (0, 3, 1, 2))
    f_cat = jnp.concatenate([f_nb, f_xyz], axis=1)
    return _att_pool(f_cat, p['att2'])


def _tail_kernel(fpc_ref, sc_ref, out_ref):
    s = fpc_ref[...] + sc_ref[...]
    out_ref[...] = jnp.where(s >= 0, s, 0.2 * s)


def _fused_tail(f_pc, sc):
    B_, C_, N_ = f_pc.shape
    return pl.pallas_call(
        _tail_kernel,
        out_shape=jax.ShapeDtypeStruct((B_, C_, N_), jnp.float32),
        grid=(B_, C_ // 8),
        in_specs=[
            pl.BlockSpec((1, 8, N_), lambda i, j: (i, j, 0)),
            pl.BlockSpec((1, 8, N_), lambda i, j: (i, j, 0)),
        ],
        out_specs=pl.BlockSpec((1, 8, N_), lambda i, j: (i, j, 0)),
    )(f_pc, sc)


def kernel(feature, xyz, neigh_idx, params):
    f_pc = _conv_bn(feature, params['mlp1'])
    f_pc = _building_block(xyz, f_pc, neigh_idx, params['lfa'])
    f_pc = _conv_bn(f_pc, params['mlp2'], act=False)
    sc = _conv_bn(feature, params['shortcut'], act=False)
    out = _fused_tail(f_pc[..., 0], sc[..., 0])
    return out[..., None]


# speedup vs baseline: 18.7578x; 18.7578x over previous
import functools

import jax
import jax.numpy as jnp
from jax import lax
from jax.experimental import pallas as pl
from jax.experimental.pallas import tpu as pltpu
from jax.experimental.pallas import tpu_sc as plsc

_B, _N, _K = 4, 50000, 16
_NK = _N * _K
_M = _B * _NK
_CH = 128
_NCHUNK = _M // _CH
_CPB = _NK // _CH
_NW = 32
_ROUNDS = -(-_NCHUNK // _NW)


def _sc_gather_body(table_hbm, idx_hbm, out_hbm, idx_v, rows_v, sem):
    wid = lax.axis_index("s") * 2 + lax.axis_index("c")

    def body(r, _):
        c = wid + r * _NW

        @pl.when(c < _NCHUNK)
        def _():
            pltpu.sync_copy(idx_hbm.at[pl.ds(c * _CH, _CH)], idx_v)
            b_off = (c // _CPB) * _N
            for t in range(_CH // 16):
                sl = pl.ds(t * 16, 16)
                idx_v[sl] = idx_v[sl] + b_off
            pltpu.async_copy(table_hbm.at[idx_v], rows_v, sem).wait()
            pltpu.sync_copy(rows_v, out_hbm.at[pl.ds(c * _CH, _CH)])

        return ()

    lax.fori_loop(0, _ROUNDS, body, ())


@functools.partial(
    pl.kernel,
    out_type=jax.ShapeDtypeStruct((_M, 16), jnp.float32),
    mesh=plsc.VectorSubcoreMesh(core_axis_name="c", subcore_axis_name="s"),
    compiler_params=pltpu.CompilerParams(use_tc_tiling_on_sc=False),
    scratch_types=[
        pltpu.VMEM((_CH,), jnp.int32),
        pltpu.VMEM((_CH, 16), jnp.float32),
        pltpu.SemaphoreType.DMA,
    ],
)
def _sc_gather(table_hbm, idx_hbm, out_hbm, idx_v, rows_v, sem):
    _sc_gather_body(table_hbm, idx_hbm, out_hbm, idx_v, rows_v, sem)


def _gather16(table, neigh_idx):
    out = _sc_gather(table.reshape(_B * _N, 16), neigh_idx.reshape(_M))
    return out.reshape(_B, _N, _K, 16)


def _conv2d(x, W, b=None):
    y = jnp.einsum('oi,binx->bonx', W, x)
    if b is not None:
        y = y + b[None, :, None, None]
    return y


def _bn(y, g, be):
    mean = y.mean(axis=(0, 2, 3), keepdims=True)
    var = y.var(axis=(0, 2, 3), keepdims=True)
    y = (y - mean) / jnp.sqrt(var + 1e-5)
    return g[None, :, None, None] * y + be[None, :, None, None]


def _conv_bn(x, p, act=True):
    y = _bn(_conv2d(x, p['W'], p['b']), p['g'], p['be'])
    return jax.nn.relu(y) if act else y


def _att_pool(fs, p):
    att = _conv2d(fs, p['Wfc'])
    scores = jax.nn.softmax(att, axis=3)
    agg = jnp.sum(fs * scores, axis=3, keepdims=True)
    return _conv_bn(agg, p['mlp'], act=True)


def _tail_kernel(fpc_ref, sc_ref, out_ref):
    s = fpc_ref[...] + sc_ref[...]
    out_ref[...] = jnp.where(s >= 0, s, 0.2 * s)


def _fused_tail(f_pc, sc):
    B_, C_, N_ = f_pc.shape
    return pl.pallas_call(
        _tail_kernel,
        out_shape=jax.ShapeDtypeStruct((B_, C_, N_), jnp.float32),
        grid=(B_, C_ // 8),
        in_specs=[
            pl.BlockSpec((1, 8, N_), lambda i, j: (i, j, 0)),
            pl.BlockSpec((1, 8, N_), lambda i, j: (i, j, 0)),
        ],
        out_specs=pl.BlockSpec((1, 8, N_), lambda i, j: (i, j, 0)),
    )(f_pc, sc)


def kernel(feature, xyz, neigh_idx, params):
    p = params
    lfa = p['lfa']
    f_pc = _conv_bn(feature, p['mlp1'])
    fp = jnp.transpose(f_pc[..., 0], (0, 2, 1))
    pad5 = jnp.zeros((_B, _N, 5), jnp.float32)
    packed = jnp.concatenate([xyz, fp, pad5], axis=-1)
    G1 = _gather16(packed, neigh_idx)
    nxyz = G1[..., :3]
    tile = jnp.broadcast_to(xyz[:, :, None, :], nxyz.shape)
    rel = tile - nxyz
    dis = jnp.sqrt(jnp.sum(rel ** 2, axis=-1, keepdims=True) + 1e-12)
    f_xyz10 = jnp.concatenate([dis, rel, tile, nxyz], axis=-1)
    f_xyz = _conv_bn(jnp.transpose(f_xyz10, (0, 3, 1, 2)), lfa['mlp1'])
    f_nb = jnp.transpose(G1[..., 3:11], (0, 3, 1, 2))
    f_cat = jnp.concatenate([f_nb, f_xyz], axis=1)
    f_agg = _att_pool(f_cat, lfa['att1'])
    f_xyz = _conv_bn(f_xyz, lfa['mlp2'])
    aggt = jnp.transpose(f_agg[..., 0], (0, 2, 1))
    pad8 = jnp.zeros((_B, _N, 8), jnp.float32)
    packed2 = jnp.concatenate([aggt, pad8], axis=-1)
    G2 = _gather16(packed2, neigh_idx)
    f_nb2 = jnp.transpose(G2[..., :8], (0, 3, 1, 2))
    f_cat = jnp.concatenate([f_nb2, f_xyz], axis=1)
    bb = _att_pool(f_cat, lfa['att2'])
    f_out = _conv_bn(bb, p['mlp2'], act=False)
    sc = _conv_bn(feature, p['shortcut'], act=False)
    out = _fused_tail(f_out[..., 0], sc[..., 0])
    return out[..., None]


# speedup vs baseline: 18.9152x; 1.0084x over previous
import functools

import jax
import jax.numpy as jnp
from jax import lax
from jax.experimental import pallas as pl
from jax.experimental.pallas import tpu as pltpu
from jax.experimental.pallas import tpu_sc as plsc

_B, _N, _K = 4, 50000, 16
_NK = _N * _K
_M = _B * _NK
_CH = 128
_NCHUNK = _M // _CH
_CPB = _NK // _CH
_NW = 32
_ROUNDS = -(-_NCHUNK // _NW)


def _sc_gather_body(table_hbm, idx_hbm, out_hbm, idx_v, rows_v, sem):
    wid = lax.axis_index("s") * 2 + lax.axis_index("c")

    def body(r, _):
        c = wid + r * _NW

        @pl.when(c < _NCHUNK)
        def _():
            pltpu.sync_copy(idx_hbm.at[pl.ds(c * _CH, _CH)], idx_v)
            b_off = (c // _CPB) * _N
            for t in range(_CH // 16):
                sl = pl.ds(t * 16, 16)
                idx_v[sl] = idx_v[sl] + b_off
            pltpu.async_copy(table_hbm.at[idx_v], rows_v, sem).wait()
            pltpu.sync_copy(rows_v, out_hbm.at[pl.ds(c * _CH, _CH)])

        return ()

    lax.fori_loop(0, _ROUNDS, body, ())


@functools.partial(
    pl.kernel,
    out_type=jax.ShapeDtypeStruct((_M, 16), jnp.float32),
    mesh=plsc.VectorSubcoreMesh(core_axis_name="c", subcore_axis_name="s"),
    compiler_params=pltpu.CompilerParams(use_tc_tiling_on_sc=False),
    scratch_types=[
        pltpu.VMEM((_CH,), jnp.int32),
        pltpu.VMEM((_CH, 16), jnp.float32),
        pltpu.SemaphoreType.DMA,
    ],
)
def _sc_gather(table_hbm, idx_hbm, out_hbm, idx_v, rows_v, sem):
    _sc_gather_body(table_hbm, idx_hbm, out_hbm, idx_v, rows_v, sem)


def _gather16(table, neigh_idx):
    out = _sc_gather(table.reshape(_B * _N, 16), neigh_idx.reshape(_M))
    return out.reshape(_B, _N, _K, 16)


def _conv2d(x, W, b=None):
    y = jnp.einsum('oi,binx->bonx', W, x)
    if b is not None:
        y = y + b[None, :, None, None]
    return y


def _bn(y, g, be):
    mean = y.mean(axis=(0, 2, 3), keepdims=True)
    var = y.var(axis=(0, 2, 3), keepdims=True)
    y = (y - mean) / jnp.sqrt(var + 1e-5)
    return g[None, :, None, None] * y + be[None, :, None, None]


def _conv_bn(x, p, act=True):
    y = _bn(_conv2d(x, p['W'], p['b']), p['g'], p['be'])
    return jax.nn.relu(y) if act else y


def _att_pool(fs, p):
    att = _conv2d(fs, p['Wfc'])
    scores = jax.nn.softmax(att, axis=3)
    agg = jnp.sum(fs * scores, axis=3, keepdims=True)
    return _conv_bn(agg, p['mlp'], act=True)


def _tail_kernel(fpc_ref, sc_ref, out_ref):
    s = fpc_ref[...] + sc_ref[...]
    out_ref[...] = jnp.where(s >= 0, s, 0.2 * s)


def _fused_tail(f_pc, sc):
    B_, C_, N_ = f_pc.shape
    return pl.pallas_call(
        _tail_kernel,
        out_shape=jax.ShapeDtypeStruct((B_, C_, N_), jnp.float32),
        grid=(B_, C_ // 8),
        in_specs=[
            pl.BlockSpec((1, 8, N_), lambda i, j: (i, j, 0)),
            pl.BlockSpec((1, 8, N_), lambda i, j: (i, j, 0)),
        ],
        out_specs=pl.BlockSpec((1, 8, N_), lambda i, j: (i, j, 0)),
    )(f_pc, sc)


def _bn_last(y, g, be):
    ax = tuple(range(y.ndim - 1))
    mean = y.mean(axis=ax, keepdims=True)
    var = y.var(axis=ax, keepdims=True)
    return g * (y - mean) / jnp.sqrt(var + 1e-5) + be


def _conv_bn_last(x, p, act=True):
    y = jnp.einsum('...i,oi->...o', x, p['W']) + p['b']
    y = _bn_last(y, p['g'], p['be'])
    return jax.nn.relu(y) if act else y


def _att_pool_last(fs, p):
    att = jnp.einsum('bnki,oi->bnko', fs, p['Wfc'])
    scores = jax.nn.softmax(att, axis=2)
    agg = jnp.sum(fs * scores, axis=2)
    return _conv_bn_last(agg, p['mlp'], act=True)


def kernel(feature, xyz, neigh_idx, params):
    p = params
    lfa = p['lfa']
    ftr = jnp.transpose(feature[..., 0], (0, 2, 1))
    f_pc = _conv_bn_last(ftr, p['mlp1'])
    pad5 = jnp.zeros((_B, _N, 5), jnp.float32)
    packed = jnp.concatenate([xyz, f_pc, pad5], axis=-1)
    G1 = _gather16(packed, neigh_idx)
    nxyz = G1[..., :3]
    f_nb = G1[..., 3:11]
    tile = jnp.broadcast_to(xyz[:, :, None, :], nxyz.shape)
    rel = tile - nxyz
    dis = jnp.sqrt(jnp.sum(rel ** 2, axis=-1, keepdims=True) + 1e-12)
    f_xyz10 = jnp.concatenate([dis, rel, tile, nxyz], axis=-1)
    f_xyz = _conv_bn_last(f_xyz10, lfa['mlp1'])
    f_cat = jnp.concatenate([f_nb, f_xyz], axis=-1)
    f_agg = _att_pool_last(f_cat, lfa['att1'])
    f_xyz = _conv_bn_last(f_xyz, lfa['mlp2'])
    pad8 = jnp.zeros((_B, _N, 8), jnp.float32)
    packed2 = jnp.concatenate([f_agg, pad8], axis=-1)
    G2 = _gather16(packed2, neigh_idx)
    f_nb2 = G2[..., :8]
    f_cat = jnp.concatenate([f_nb2, f_xyz], axis=-1)
    bb = _att_pool_last(f_cat, lfa['att2'])
    f_out = _conv_bn_last(bb, p['mlp2'], act=False)
    sc = _conv_bn_last(ftr, p['shortcut'], act=False)
    out = _fused_tail(
        jnp.transpose(f_out, (0, 2, 1)), jnp.transpose(sc, (0, 2, 1)))
    return out[..., None]
